# Initial kernel scaffold; baseline (speedup 1.0000x reference)
import functools

import jax
import jax.numpy as jnp
from jax import lax
from jax.experimental import pallas as pl
from jax.experimental.pallas import tpu as pltpu
from jax.experimental.pallas import tpu_sc as plsc

NC = 2
NS = 16
NW = NC * NS
TW = 32
CH = 80


def _table_body(nf_ref, m1_ref, m2_ref, m3_ref, t1_ref, t2_ref):
    x = nf_ref[...]
    x2 = x * x
    qpart = jnp.dot(x2, m2_ref[...], preferred_element_type=jnp.float32)
    t1_ref[...] = jnp.dot(x, m1_ref[...], preferred_element_type=jnp.float32) + qpart
    t2_ref[...] = jnp.dot(x, m3_ref[...], preferred_element_type=jnp.float32) + qpart


def _gather_body(per_w, n_ch, tsrc, tdst, ii, jj, out,
                 ivm, jvm, ba0, ba1, bb0, bb1, bw0, bw1,
                 sa0, sa1, sb0, sb1, sw0, sw1):
    wid = lax.axis_index("s") * NC + lax.axis_index("c")
    base = wid * per_w
    pltpu.sync_copy(ii.at[pl.ds(base, per_w)], ivm)
    pltpu.sync_copy(jj.at[pl.ds(base, per_w)], jvm)

    bufs = ((ba0, bb0, bw0, sa0, sb0, sw0), (ba1, bb1, bw1, sa1, sb1, sw1))

    def start(c, slot):
        ba, bb, _, sa, sb, _ = bufs[slot]
        off = pl.multiple_of(c * CH, 8)
        pltpu.async_copy(tsrc.at[ivm.at[pl.ds(off, CH)]], ba, sa)
        pltpu.async_copy(tdst.at[jvm.at[pl.ds(off, CH)]], bb, sb)

    def process(c, slot):
        ba, bb, bw, sa, sb, sw = bufs[slot]
        off = pl.multiple_of(c * CH, 8)
        pltpu.make_async_copy(tsrc.at[ivm.at[pl.ds(off, CH)]], ba, sa).wait()
        pltpu.make_async_copy(tdst.at[jvm.at[pl.ds(off, CH)]], bb, sb).wait()

        @pl.when(c >= 2)
        def _():
            pltpu.make_async_copy(bw, out.at[pl.ds(base + off, CH)], sw).wait()

        for rr in range(CH):
            for hh in range(TW // 16):
                sl = pl.ds(hh * 16, 16)
                bw[rr, sl] = ba[rr, sl] + bb[rr, sl]
        pltpu.async_copy(bw, out.at[pl.ds(base + off, CH)], sw)

    start(0, 0)

    def body(c2, carry):
        c0 = 2 * c2
        c1 = c0 + 1

        @pl.when(c1 < n_ch)
        def _():
            start(c1, 1)

        process(c0, 0)

        @pl.when(c0 + 2 < n_ch)
        def _():
            start(c0 + 2, 0)

        @pl.when(c1 < n_ch)
        def _():
            process(c1, 1)

        return carry

    lax.fori_loop(0, (n_ch + 1) // 2, body, 0)

    last0 = (n_ch - 1) // 2 * 2 * CH
    pltpu.make_async_copy(bw0, out.at[pl.ds(base + last0, CH)], sw0).wait()
    if n_ch % 2 == 0:
        last1 = (n_ch - 1) * CH
        pltpu.make_async_copy(bw1, out.at[pl.ds(base + last1, CH)], sw1).wait()
    elif n_ch > 1:
        last1 = (n_ch - 2) * CH
        pltpu.make_async_copy(bw1, out.at[pl.ds(base + last1, CH)], sw1).wait()


def _mlp_body(cat_dim, g_ref, e_ref, c_ref, w2_ref, uvb_ref, o_ref):
    gb = g_ref[...]
    e = e_ref[...]
    p = gb[:, 0:16]
    ssum = gb[:, 16:17]
    qsum = gb[:, 17:18]
    se = jnp.sum(e, axis=1, keepdims=True)
    qe = jnp.sum(e * e, axis=1, keepdims=True)
    scale = 1.0 / cat_dim
    mu = (ssum + se) * scale
    var = (qsum + qe) * scale - mu * mu
    inv = lax.rsqrt(var + 1e-5)
    ec = jnp.dot(e, c_ref[...], preferred_element_type=jnp.float32)
    u = uvb_ref[0:1, :]
    v = uvb_ref[1:2, :]
    b2r = uvb_ref[2:3, :]
    y1 = (p + ec) * inv - (mu * inv) * u + v
    y1 = jnp.where(y1 > 0, y1, 0.01 * y1)
    y2 = jnp.dot(y1, w2_ref[...], preferred_element_type=jnp.float32) + b2r
    o_ref[...] = e + y2


def kernel(h0, edge_index, edge_w, ln_g, ln_b, W1, b1, W2, b2, r, basis):
    N, D, _ = h0.shape
    E = edge_index.shape[1]
    ED = edge_w.shape[1]
    cat_dim = 2 * D + ED
    nf = jnp.squeeze(h0, axis=-1)

    w1g = W1 * ln_g[:, None]
    a_m = w1g[:D]
    b_m = w1g[D:2 * D]
    c_m = w1g[2 * D:]
    u = jnp.sum(w1g, axis=0)
    v = ln_b @ W1 + b1
    zcol = jnp.zeros((D, TW - 18), jnp.float32)
    one = jnp.ones((D, 1), jnp.float32)
    zero = jnp.zeros((D, 1), jnp.float32)
    m1 = jnp.concatenate([a_m, one, zero, zcol], axis=1)
    m3 = jnp.concatenate([b_m, one, zero, zcol], axis=1)
    m2 = jnp.concatenate([jnp.zeros((D, 16), jnp.float32), zero, one, zcol], axis=1)
    uvb = jnp.concatenate([u[None, :], v[None, :], b2[None, :],
                           jnp.zeros((5, ED), jnp.float32)], axis=0)

    nb = 5
    bn = N // nb
    t_src, t_dst = pl.pallas_call(
        _table_body,
        grid=(nb,),
        in_specs=[
            pl.BlockSpec((bn, D), lambda b: (b, 0)),
            pl.BlockSpec((D, TW), lambda b: (0, 0)),
            pl.BlockSpec((D, TW), lambda b: (0, 0)),
            pl.BlockSpec((D, TW), lambda b: (0, 0)),
        ],
        out_specs=[
            pl.BlockSpec((bn, TW), lambda b: (b, 0)),
            pl.BlockSpec((bn, TW), lambda b: (b, 0)),
        ],
        out_shape=[
            jax.ShapeDtypeStruct((N, TW), jnp.float32),
            jax.ShapeDtypeStruct((N, TW), jnp.float32),
        ],
    )(nf, m1, m2, m3)

    per_w = E // NW
    n_ch = per_w // CH
    mesh = plsc.VectorSubcoreMesh(core_axis_name="c", subcore_axis_name="s")
    gather_fn = pl.kernel(
        functools.partial(_gather_body, per_w, n_ch),
        mesh=mesh,
        out_type=jax.ShapeDtypeStruct((E, TW), jnp.float32),
        scratch_types=[
            pltpu.VMEM((per_w,), jnp.int32),
            pltpu.VMEM((per_w,), jnp.int32),
            pltpu.VMEM((CH, TW), jnp.float32),
            pltpu.VMEM((CH, TW), jnp.float32),
            pltpu.VMEM((CH, TW), jnp.float32),
            pltpu.VMEM((CH, TW), jnp.float32),
            pltpu.VMEM((CH, TW), jnp.float32),
            pltpu.VMEM((CH, TW), jnp.float32),
            pltpu.SemaphoreType.DMA,
            pltpu.SemaphoreType.DMA,
            pltpu.SemaphoreType.DMA,
            pltpu.SemaphoreType.DMA,
            pltpu.SemaphoreType.DMA,
            pltpu.SemaphoreType.DMA,
        ],
    )
    g_arr = gather_fn(t_src, t_dst, edge_index[0], edge_index[1])

    be = 8000
    ne = E // be
    out = pl.pallas_call(
        functools.partial(_mlp_body, float(cat_dim)),
        grid=(ne,),
        in_specs=[
            pl.BlockSpec((be, TW), lambda b: (b, 0)),
            pl.BlockSpec((be, ED), lambda b: (b, 0)),
            pl.BlockSpec((ED, ED), lambda b: (0, 0)),
            pl.BlockSpec((ED, ED), lambda b: (0, 0)),
            pl.BlockSpec((8, ED), lambda b: (0, 0)),
        ],
        out_specs=pl.BlockSpec((be, ED), lambda b: (b, 0)),
        out_shape=jax.ShapeDtypeStruct((E, ED), jnp.float32),
    )(g_arr, edge_w, c_m, W2, uvb)
    return out


# speedup vs baseline: 3.4955x; 3.4955x over previous
import functools

import jax
import jax.numpy as jnp
from jax import lax
from jax.experimental import pallas as pl
from jax.experimental.pallas import tpu as pltpu
from jax.experimental.pallas import tpu_sc as plsc

NC = 2
NS = 16
NW = NC * NS
TW = 32
CH = 80


def _table_body(nf_ref, m1_ref, m2_ref, m3_ref, t1_ref, t2_ref):
    x = nf_ref[...]
    x2 = x * x
    qpart = jnp.dot(x2, m2_ref[...], preferred_element_type=jnp.float32)
    t1_ref[...] = jnp.dot(x, m1_ref[...], preferred_element_type=jnp.float32) + qpart
    t2_ref[...] = jnp.dot(x, m3_ref[...], preferred_element_type=jnp.float32) + qpart


def _gather_body(per_w, n_ch, tsrc, tdst, ii, jj, out,
                 ivm, jvm, ba0, ba1, bb0, bb1, bw0, bw1,
                 sa0, sa1, sb0, sb1, sw0, sw1):
    wid = lax.axis_index("s") * NC + lax.axis_index("c")
    base = wid * per_w
    pltpu.sync_copy(ii.at[wid], ivm)
    pltpu.sync_copy(jj.at[wid], jvm)

    bufs = ((ba0, bb0, bw0, sa0, sb0, sw0), (ba1, bb1, bw1, sa1, sb1, sw1))

    def start(c, slot):
        ba, bb, _, sa, sb, _ = bufs[slot]
        pltpu.async_copy(tsrc.at[ivm.at[c]], ba, sa)
        pltpu.async_copy(tdst.at[jvm.at[c]], bb, sb)

    def process(c, slot):
        ba, bb, bw, sa, sb, sw = bufs[slot]
        off = pl.multiple_of(c * CH, 8)
        pltpu.make_async_copy(tsrc.at[ivm.at[c]], ba, sa).wait()
        pltpu.make_async_copy(tdst.at[jvm.at[c]], bb, sb).wait()

        @pl.when(c >= 2)
        def _():
            pltpu.make_async_copy(bw, out.at[pl.ds(base + off, CH)], sw).wait()

        for rr in range(CH):
            for hh in range(TW // 16):
                sl = pl.ds(hh * 16, 16)
                bw[rr, sl] = ba[rr, sl] + bb[rr, sl]
        pltpu.async_copy(bw, out.at[pl.ds(base + off, CH)], sw)

    start(0, 0)

    def body(c2, carry):
        c0 = 2 * c2
        c1 = c0 + 1

        @pl.when(c1 < n_ch)
        def _():
            start(c1, 1)

        process(c0, 0)

        @pl.when(c0 + 2 < n_ch)
        def _():
            start(c0 + 2, 0)

        @pl.when(c1 < n_ch)
        def _():
            process(c1, 1)

        return carry

    lax.fori_loop(0, (n_ch + 1) // 2, body, 0)

    last0 = (n_ch - 1) // 2 * 2 * CH
    pltpu.make_async_copy(bw0, out.at[pl.ds(base + last0, CH)], sw0).wait()
    if n_ch % 2 == 0:
        last1 = (n_ch - 1) * CH
        pltpu.make_async_copy(bw1, out.at[pl.ds(base + last1, CH)], sw1).wait()
    elif n_ch > 1:
        last1 = (n_ch - 2) * CH
        pltpu.make_async_copy(bw1, out.at[pl.ds(base + last1, CH)], sw1).wait()


def _mlp_body(cat_dim, g_ref, e_ref, c_ref, w2_ref, uvb_ref, o_ref):
    gb = g_ref[...]
    e = e_ref[...]
    p = gb[:, 0:16]
    ssum = gb[:, 16:17]
    qsum = gb[:, 17:18]
    se = jnp.sum(e, axis=1, keepdims=True)
    qe = jnp.sum(e * e, axis=1, keepdims=True)
    scale = 1.0 / cat_dim
    mu = (ssum + se) * scale
    var = (qsum + qe) * scale - mu * mu
    inv = lax.rsqrt(var + 1e-5)
    ec = jnp.dot(e, c_ref[...], preferred_element_type=jnp.float32)
    u = uvb_ref[0:1, :]
    v = uvb_ref[1:2, :]
    b2r = uvb_ref[2:3, :]
    y1 = (p + ec) * inv - (mu * inv) * u + v
    y1 = jnp.where(y1 > 0, y1, 0.01 * y1)
    y2 = jnp.dot(y1, w2_ref[...], preferred_element_type=jnp.float32) + b2r
    o_ref[...] = e + y2


def kernel(h0, edge_index, edge_w, ln_g, ln_b, W1, b1, W2, b2, r, basis):
    N, D, _ = h0.shape
    E = edge_index.shape[1]
    ED = edge_w.shape[1]
    cat_dim = 2 * D + ED
    nf = jnp.squeeze(h0, axis=-1)

    w1g = W1 * ln_g[:, None]
    a_m = w1g[:D]
    b_m = w1g[D:2 * D]
    c_m = w1g[2 * D:]
    u = jnp.sum(w1g, axis=0)
    v = ln_b @ W1 + b1
    zcol = jnp.zeros((D, TW - 18), jnp.float32)
    one = jnp.ones((D, 1), jnp.float32)
    zero = jnp.zeros((D, 1), jnp.float32)
    m1 = jnp.concatenate([a_m, one, zero, zcol], axis=1)
    m3 = jnp.concatenate([b_m, one, zero, zcol], axis=1)
    m2 = jnp.concatenate([jnp.zeros((D, 16), jnp.float32), zero, one, zcol], axis=1)
    uvb = jnp.concatenate([u[None, :], v[None, :], b2[None, :],
                           jnp.zeros((5, ED), jnp.float32)], axis=0)

    nb = 5
    bn = N // nb
    t_src, t_dst = pl.pallas_call(
        _table_body,
        grid=(nb,),
        in_specs=[
            pl.BlockSpec((bn, D), lambda b: (b, 0)),
            pl.BlockSpec((D, TW), lambda b: (0, 0)),
            pl.BlockSpec((D, TW), lambda b: (0, 0)),
            pl.BlockSpec((D, TW), lambda b: (0, 0)),
        ],
        out_specs=[
            pl.BlockSpec((bn, TW), lambda b: (b, 0)),
            pl.BlockSpec((bn, TW), lambda b: (b, 0)),
        ],
        out_shape=[
            jax.ShapeDtypeStruct((N, TW), jnp.float32),
            jax.ShapeDtypeStruct((N, TW), jnp.float32),
        ],
    )(nf, m1, m2, m3)

    per_w = E // NW
    n_ch = per_w // CH
    mesh = plsc.VectorSubcoreMesh(core_axis_name="c", subcore_axis_name="s",
                                  num_cores=NC, num_subcores=NS)
    gather_fn = pl.kernel(
        functools.partial(_gather_body, per_w, n_ch),
        mesh=mesh,
        compiler_params=pltpu.CompilerParams(use_tc_tiling_on_sc=False),
        out_type=jax.ShapeDtypeStruct((E, TW), jnp.float32),
        scratch_types=[
            pltpu.VMEM((n_ch, CH), jnp.int32),
            pltpu.VMEM((n_ch, CH), jnp.int32),
            pltpu.VMEM((CH, TW), jnp.float32),
            pltpu.VMEM((CH, TW), jnp.float32),
            pltpu.VMEM((CH, TW), jnp.float32),
            pltpu.VMEM((CH, TW), jnp.float32),
            pltpu.VMEM((CH, TW), jnp.float32),
            pltpu.VMEM((CH, TW), jnp.float32),
            pltpu.SemaphoreType.DMA,
            pltpu.SemaphoreType.DMA,
            pltpu.SemaphoreType.DMA,
            pltpu.SemaphoreType.DMA,
            pltpu.SemaphoreType.DMA,
            pltpu.SemaphoreType.DMA,
        ],
    )
    i3 = edge_index[0].reshape(NW, n_ch, CH)
    j3 = edge_index[1].reshape(NW, n_ch, CH)
    g_arr = gather_fn(t_src, t_dst, i3, j3)

    be = 8000
    ne = E // be
    out = pl.pallas_call(
        functools.partial(_mlp_body, float(cat_dim)),
        grid=(ne,),
        in_specs=[
            pl.BlockSpec((be, TW), lambda b: (b, 0)),
            pl.BlockSpec((be, ED), lambda b: (b, 0)),
            pl.BlockSpec((ED, ED), lambda b: (0, 0)),
            pl.BlockSpec((ED, ED), lambda b: (0, 0)),
            pl.BlockSpec((8, ED), lambda b: (0, 0)),
        ],
        out_specs=pl.BlockSpec((be, ED), lambda b: (b, 0)),
        out_shape=jax.ShapeDtypeStruct((E, ED), jnp.float32),
    )(g_arr, edge_w, c_m, W2, uvb)
    return out


# speedup vs baseline: 6.2259x; 1.7811x over previous
import functools

import jax
import jax.numpy as jnp
from jax import lax
from jax.experimental import pallas as pl
from jax.experimental.pallas import tpu as pltpu
from jax.experimental.pallas import tpu_sc as plsc

NC = 2
NS = 16
NW = NC * NS
TW = 32
CH = 80


def _table_body(nf_ref, m1_ref, m2_ref, m3_ref, t1_ref, t2_ref):
    x = nf_ref[...]
    x2 = x * x
    qpart = jnp.dot(x2, m2_ref[...], preferred_element_type=jnp.float32)
    t1_ref[...] = jnp.dot(x, m1_ref[...], preferred_element_type=jnp.float32) + qpart
    t2_ref[...] = jnp.dot(x, m3_ref[...], preferred_element_type=jnp.float32) + qpart


def _gather_body(per_w, n_ch, tsrc, tdst, ii, jj, g1o, g2o,
                 ivm, jvm, ba0, ba1, bb0, bb1, bw0, bw1, bp0, bp1,
                 sa0, sa1, sb0, sb1, sw0, sw1, sp0, sp1):
    wid = lax.axis_index("s") * NC + lax.axis_index("c")
    base = wid * per_w
    base2 = wid * (per_w // 8)
    pltpu.sync_copy(ii.at[wid], ivm)
    pltpu.sync_copy(jj.at[wid], jvm)

    rows_half = lax.shift_right_logical(lax.iota(jnp.int32, 16), 1)
    cols_sq = 16 + lax.bitwise_and(lax.iota(jnp.int32, 16), 1)

    bufs = ((ba0, bb0, bw0, bp0, sa0, sb0, sw0, sp0),
            (ba1, bb1, bw1, bp1, sa1, sb1, sw1, sp1))

    def start(c, slot):
        ba, bb = bufs[slot][0], bufs[slot][1]
        sa, sb = bufs[slot][4], bufs[slot][5]
        pltpu.async_copy(tsrc.at[ivm.at[c]], ba, sa)
        pltpu.async_copy(tdst.at[jvm.at[c]], bb, sb)

    def process(c, slot):
        ba, bb, bw, bp, sa, sb, sw, sp = bufs[slot]
        off = pl.multiple_of(c * CH, 8)
        off2 = c * (CH // 8)
        pltpu.make_async_copy(tsrc.at[ivm.at[c]], ba, sa).wait()
        pltpu.make_async_copy(tdst.at[jvm.at[c]], bb, sb).wait()

        @pl.when(c >= 2)
        def _():
            pltpu.make_async_copy(bw, g1o.at[pl.ds(base + off, CH)], sw).wait()
            pltpu.make_async_copy(bp, g2o.at[pl.ds(base2 + off2, CH // 8)], sp).wait()

        for rr in range(CH):
            bw[rr, :] = ba[rr, pl.ds(0, 16)] + bb[rr, pl.ds(0, 16)]
        for pp in range(CH // 8):
            rows = rows_half + (8 * pp)
            va = plsc.load_gather(ba, [rows, cols_sq])
            vb = plsc.load_gather(bb, [rows, cols_sq])
            bp[pp, :] = va + vb
        pltpu.async_copy(bw, g1o.at[pl.ds(base + off, CH)], sw)
        pltpu.async_copy(bp, g2o.at[pl.ds(base2 + off2, CH // 8)], sp)

    start(0, 0)

    def body(c2, carry):
        c0 = 2 * c2
        c1 = c0 + 1

        @pl.when(c1 < n_ch)
        def _():
            start(c1, 1)

        process(c0, 0)

        @pl.when(c0 + 2 < n_ch)
        def _():
            start(c0 + 2, 0)

        @pl.when(c1 < n_ch)
        def _():
            process(c1, 1)

        return carry

    lax.fori_loop(0, (n_ch + 1) // 2, body, 0)

    last0 = (n_ch - 1) // 2 * 2
    pltpu.make_async_copy(bw0, g1o.at[pl.ds(base + last0 * CH, CH)], sw0).wait()
    pltpu.make_async_copy(bp0, g2o.at[pl.ds(base2 + last0 * (CH // 8), CH // 8)], sp0).wait()
    last1 = (n_ch - 1) if n_ch % 2 == 0 else (n_ch - 2)
    if last1 >= 1:
        pltpu.make_async_copy(bw1, g1o.at[pl.ds(base + last1 * CH, CH)], sw1).wait()
        pltpu.make_async_copy(bp1, g2o.at[pl.ds(base2 + last1 * (CH // 8), CH // 8)], sp1).wait()


def _mlp_body(cat_dim, e_ref, g1_ref, g2_ref, s1_ref, me_ref, cw_ref, uvb_ref, o_ref):
    e = e_ref[...]
    g1 = g1_ref[...]
    g2 = g2_ref[...]
    s1 = s1_ref[...]
    me = me_ref[...]
    cbd = cw_ref[:, 0:128]
    w2bd = cw_ref[:, 128:256]
    scale = 1.0 / cat_dim
    se_b = jnp.dot(e, s1, preferred_element_type=jnp.float32)
    qe_b = jnp.dot(e * e, s1, preferred_element_type=jnp.float32)
    stats = jnp.dot(g2, me, preferred_element_type=jnp.float32)
    mu = (stats[:, 0:128] + se_b) * scale
    var = (stats[:, 128:256] + qe_b) * scale - mu * mu
    inv = lax.rsqrt(var + 1e-5)
    ec = jnp.dot(e, cbd, preferred_element_type=jnp.float32)
    u = uvb_ref[0:1, :]
    v = uvb_ref[1:2, :]
    b2r = uvb_ref[2:3, :]
    y1 = (g1 + ec) * inv - (mu * inv) * u + v
    y1 = jnp.where(y1 > 0, y1, 0.01 * y1)
    y2 = jnp.dot(y1, w2bd, preferred_element_type=jnp.float32) + b2r
    o_ref[...] = e + y2


def kernel(h0, edge_index, edge_w, ln_g, ln_b, W1, b1, W2, b2, r, basis):
    N, D, _ = h0.shape
    E = edge_index.shape[1]
    ED = edge_w.shape[1]
    cat_dim = 2 * D + ED
    nf = jnp.squeeze(h0, axis=-1)

    w1g = W1 * ln_g[:, None]
    a_m = w1g[:D]
    b_m = w1g[D:2 * D]
    c_m = w1g[2 * D:]
    u = jnp.sum(w1g, axis=0)
    v = ln_b @ W1 + b1
    zcol = jnp.zeros((D, TW - 18), jnp.float32)
    one = jnp.ones((D, 1), jnp.float32)
    zero = jnp.zeros((D, 1), jnp.float32)
    m1 = jnp.concatenate([a_m, one, zero, zcol], axis=1)
    m3 = jnp.concatenate([b_m, one, zero, zcol], axis=1)
    m2 = jnp.concatenate([jnp.zeros((D, 16), jnp.float32), zero, one, zcol], axis=1)

    gid = jnp.arange(128) // ED
    s1c = (gid[:, None] == gid[None, :]).astype(jnp.float32)
    lane16 = jnp.arange(16)
    mexp_s = ((lane16[:, None] == 2 * gid[None, :])).astype(jnp.float32)
    mexp_q = ((lane16[:, None] == 2 * gid[None, :] + 1)).astype(jnp.float32)
    mec = jnp.concatenate([mexp_s, mexp_q], axis=1)
    blk = (gid[:, None] == gid[None, :])
    cbd = jnp.where(blk, jnp.tile(c_m, (8, 8)), 0.0)
    w2bd = jnp.where(blk, jnp.tile(W2, (8, 8)), 0.0)
    cw = jnp.concatenate([cbd, w2bd], axis=1)
    uvb = jnp.concatenate([jnp.tile(u, 8)[None, :], jnp.tile(v, 8)[None, :],
                           jnp.tile(b2, 8)[None, :],
                           jnp.zeros((5, 128), jnp.float32)], axis=0)

    nb = 5
    bn = N // nb
    t_src, t_dst = pl.pallas_call(
        _table_body,
        grid=(nb,),
        in_specs=[
            pl.BlockSpec((bn, D), lambda b: (b, 0)),
            pl.BlockSpec((D, TW), lambda b: (0, 0)),
            pl.BlockSpec((D, TW), lambda b: (0, 0)),
            pl.BlockSpec((D, TW), lambda b: (0, 0)),
        ],
        out_specs=[
            pl.BlockSpec((bn, TW), lambda b: (b, 0)),
            pl.BlockSpec((bn, TW), lambda b: (b, 0)),
        ],
        out_shape=[
            jax.ShapeDtypeStruct((N, TW), jnp.float32),
            jax.ShapeDtypeStruct((N, TW), jnp.float32),
        ],
    )(nf, m1, m2, m3)

    per_w = E // NW
    n_ch = per_w // CH
    mesh = plsc.VectorSubcoreMesh(core_axis_name="c", subcore_axis_name="s",
                                  num_cores=NC, num_subcores=NS)
    gather_fn = pl.kernel(
        functools.partial(_gather_body, per_w, n_ch),
        mesh=mesh,
        compiler_params=pltpu.CompilerParams(use_tc_tiling_on_sc=False,
                                             needs_layout_passes=False),
        out_type=[
            jax.ShapeDtypeStruct((E, ED), jnp.float32),
            jax.ShapeDtypeStruct((E // 8, ED), jnp.float32),
        ],
        scratch_types=[
            pltpu.VMEM((n_ch, CH), jnp.int32),
            pltpu.VMEM((n_ch, CH), jnp.int32),
            pltpu.VMEM((CH, TW), jnp.float32),
            pltpu.VMEM((CH, TW), jnp.float32),
            pltpu.VMEM((CH, TW), jnp.float32),
            pltpu.VMEM((CH, TW), jnp.float32),
            pltpu.VMEM((CH, ED), jnp.float32),
            pltpu.VMEM((CH, ED), jnp.float32),
            pltpu.VMEM((CH // 8, ED), jnp.float32),
            pltpu.VMEM((CH // 8, ED), jnp.float32),
            pltpu.SemaphoreType.DMA,
            pltpu.SemaphoreType.DMA,
            pltpu.SemaphoreType.DMA,
            pltpu.SemaphoreType.DMA,
            pltpu.SemaphoreType.DMA,
            pltpu.SemaphoreType.DMA,
            pltpu.SemaphoreType.DMA,
            pltpu.SemaphoreType.DMA,
        ],
    )
    i3 = edge_index[0].reshape(NW, n_ch, CH)
    j3 = edge_index[1].reshape(NW, n_ch, CH)
    g1, g2p = gather_fn(t_src, t_dst, i3, j3)

    ew = edge_w.reshape(E // 8, 8 * ED)
    g1w = g1.reshape(E // 8, 8 * ED)
    rw = 4000
    ne = (E // 8) // rw
    outw = pl.pallas_call(
        functools.partial(_mlp_body, float(cat_dim)),
        grid=(ne,),
        in_specs=[
            pl.BlockSpec((rw, 128), lambda b: (b, 0)),
            pl.BlockSpec((rw, 128), lambda b: (b, 0)),
            pl.BlockSpec((rw, ED), lambda b: (b, 0)),
            pl.BlockSpec((128, 128), lambda b: (0, 0)),
            pl.BlockSpec((16, 256), lambda b: (0, 0)),
            pl.BlockSpec((128, 256), lambda b: (0, 0)),
            pl.BlockSpec((8, 128), lambda b: (0, 0)),
        ],
        out_specs=pl.BlockSpec((rw, 128), lambda b: (b, 0)),
        out_shape=jax.ShapeDtypeStruct((E // 8, 8 * ED), jnp.float32),
    )(ew, g1w, g2p, s1c, mec, cw, uvb)
    return outw.reshape(E, ED)


# speedup vs baseline: 10.3630x; 1.6645x over previous
import functools

import jax
import jax.numpy as jnp
from jax import lax
from jax.experimental import pallas as pl
from jax.experimental.pallas import tpu as pltpu
from jax.experimental.pallas import tpu_sc as plsc

NC = 2
NS = 16
NW = NC * NS
TW = 32
CH = 80


def _table_body(nf_ref, m1_ref, m2_ref, m3_ref, t1_ref, t2_ref):
    x = nf_ref[...]
    x2 = x * x
    qpart = jnp.dot(x2, m2_ref[...], preferred_element_type=jnp.float32)
    t1_ref[...] = jnp.dot(x, m1_ref[...], preferred_element_type=jnp.float32) + qpart
    t2_ref[...] = jnp.dot(x, m3_ref[...], preferred_element_type=jnp.float32) + qpart


def _gather_body(per_w, n_ch, tsrc, tdst, ii, jj, gx,
                 ivm, jvm, ba0, ba1, bb0, bb1, bw0, bw1,
                 sa0, sa1, sb0, sb1, sw0, sw1):
    wid = lax.axis_index("s") * NC + lax.axis_index("c")
    base = wid * per_w
    pltpu.sync_copy(ii.at[wid], ivm)
    pltpu.sync_copy(jj.at[wid], jvm)

    lanes = lax.iota(jnp.int32, 16)
    c16 = jnp.full((16,), 16, jnp.int32)
    c17 = jnp.full((16,), 17, jnp.int32)

    bufs = ((ba0, bb0, bw0, sa0, sb0, sw0),
            (ba1, bb1, bw1, sa1, sb1, sw1))

    def start(c, slot):
        ba, bb = bufs[slot][0], bufs[slot][1]
        sa, sb = bufs[slot][3], bufs[slot][4]
        pltpu.async_copy(tsrc.at[ivm.at[c]], ba, sa)
        pltpu.async_copy(tdst.at[jvm.at[c]], bb, sb)

    def process(c, slot):
        ba, bb, bw, sa, sb, sw = bufs[slot]
        off = pl.multiple_of(c * CH, 8)
        pltpu.make_async_copy(tsrc.at[ivm.at[c]], ba, sa).wait()
        pltpu.make_async_copy(tdst.at[jvm.at[c]], bb, sb).wait()

        @pl.when(c >= 2)
        def _():
            pltpu.make_async_copy(
                bw, gx.at[pl.ds(0, 18), pl.ds(base + off, CH)], sw).wait()

        for rr in range(CH):
            val = ba[rr, pl.ds(0, 16)] + bb[rr, pl.ds(0, 16)]
            plsc.store_scatter(bw, [lanes, jnp.full((16,), rr, jnp.int32)], val)
        for pp in range(CH // 16):
            rows = lanes + (16 * pp)
            sl = pl.ds(16 * pp, 16)
            bw[16, sl] = (plsc.load_gather(ba, [rows, c16])
                          + plsc.load_gather(bb, [rows, c16]))
            bw[17, sl] = (plsc.load_gather(ba, [rows, c17])
                          + plsc.load_gather(bb, [rows, c17]))
        pltpu.async_copy(bw, gx.at[pl.ds(0, 18), pl.ds(base + off, CH)], sw)

    start(0, 0)

    def body(c2, carry):
        c0 = 2 * c2
        c1 = c0 + 1

        @pl.when(c1 < n_ch)
        def _():
            start(c1, 1)

        process(c0, 0)

        @pl.when(c0 + 2 < n_ch)
        def _():
            start(c0 + 2, 0)

        @pl.when(c1 < n_ch)
        def _():
            process(c1, 1)

        return carry

    lax.fori_loop(0, (n_ch + 1) // 2, body, 0)

    last0 = (n_ch - 1) // 2 * 2
    pltpu.make_async_copy(
        bw0, gx.at[pl.ds(0, 18), pl.ds(base + last0 * CH, CH)], sw0).wait()
    last1 = (n_ch - 1) if n_ch % 2 == 0 else (n_ch - 2)
    if last1 >= 1:
        pltpu.make_async_copy(
            bw1, gx.at[pl.ds(0, 18), pl.ds(base + last1 * CH, CH)], sw1).wait()


def _mlp_body(cat_dim, e_ref, gx_ref, cw_ref, uvb_ref, o_ref):
    et = e_ref[...]
    g1t = gx_ref[0:16, :]
    srow = gx_ref[16:17, :]
    qrow = gx_ref[17:18, :]
    ct = cw_ref[:, 0:16]
    w2t = cw_ref[:, 16:32]
    scale = 1.0 / cat_dim
    se = jnp.sum(et, axis=0, keepdims=True)
    qe = jnp.sum(et * et, axis=0, keepdims=True)
    mu = (srow + se) * scale
    var = (qrow + qe) * scale - mu * mu
    inv = lax.rsqrt(var + 1e-5)
    ect = jnp.dot(ct, et, preferred_element_type=jnp.float32)
    u = uvb_ref[:, 0:1]
    v = uvb_ref[:, 1:2]
    b2c = uvb_ref[:, 2:3]
    y1 = (g1t + ect) * inv - u * (mu * inv) + v
    y1 = jnp.where(y1 > 0, y1, 0.01 * y1)
    y2 = jnp.dot(w2t, y1, preferred_element_type=jnp.float32) + b2c
    o_ref[...] = et + y2


def kernel(h0, edge_index, edge_w, ln_g, ln_b, W1, b1, W2, b2, r, basis):
    N, D, _ = h0.shape
    E = edge_index.shape[1]
    ED = edge_w.shape[1]
    cat_dim = 2 * D + ED
    nf = jnp.squeeze(h0, axis=-1)

    w1g = W1 * ln_g[:, None]
    a_m = w1g[:D]
    b_m = w1g[D:2 * D]
    c_m = w1g[2 * D:]
    u = jnp.sum(w1g, axis=0)
    v = ln_b @ W1 + b1
    zcol = jnp.zeros((D, TW - 18), jnp.float32)
    one = jnp.ones((D, 1), jnp.float32)
    zero = jnp.zeros((D, 1), jnp.float32)
    m1 = jnp.concatenate([a_m, one, zero, zcol], axis=1)
    m3 = jnp.concatenate([b_m, one, zero, zcol], axis=1)
    m2 = jnp.concatenate([jnp.zeros((D, 16), jnp.float32), zero, one, zcol], axis=1)

    cw = jnp.concatenate([c_m.T, W2.T], axis=1)
    uvb = jnp.concatenate([u[:, None], v[:, None], b2[:, None],
                           jnp.zeros((ED, 5), jnp.float32)], axis=1)

    nb = 5
    bn = N // nb
    t_src, t_dst = pl.pallas_call(
        _table_body,
        grid=(nb,),
        in_specs=[
            pl.BlockSpec((bn, D), lambda b: (b, 0)),
            pl.BlockSpec((D, TW), lambda b: (0, 0)),
            pl.BlockSpec((D, TW), lambda b: (0, 0)),
            pl.BlockSpec((D, TW), lambda b: (0, 0)),
        ],
        out_specs=[
            pl.BlockSpec((bn, TW), lambda b: (b, 0)),
            pl.BlockSpec((bn, TW), lambda b: (b, 0)),
        ],
        out_shape=[
            jax.ShapeDtypeStruct((N, TW), jnp.float32),
            jax.ShapeDtypeStruct((N, TW), jnp.float32),
        ],
    )(nf, m1, m2, m3)

    per_w = E // NW
    n_ch = per_w // CH
    mesh = plsc.VectorSubcoreMesh(core_axis_name="c", subcore_axis_name="s",
                                  num_cores=NC, num_subcores=NS)
    gather_fn = pl.kernel(
        functools.partial(_gather_body, per_w, n_ch),
        mesh=mesh,
        compiler_params=pltpu.CompilerParams(use_tc_tiling_on_sc=False,
                                             needs_layout_passes=False),
        out_type=jax.ShapeDtypeStruct((24, E), jnp.float32),
        scratch_types=[
            pltpu.VMEM((n_ch, CH), jnp.int32),
            pltpu.VMEM((n_ch, CH), jnp.int32),
            pltpu.VMEM((CH, TW), jnp.float32),
            pltpu.VMEM((CH, TW), jnp.float32),
            pltpu.VMEM((CH, TW), jnp.float32),
            pltpu.VMEM((CH, TW), jnp.float32),
            pltpu.VMEM((18, CH), jnp.float32),
            pltpu.VMEM((18, CH), jnp.float32),
            pltpu.SemaphoreType.DMA,
            pltpu.SemaphoreType.DMA,
            pltpu.SemaphoreType.DMA,
            pltpu.SemaphoreType.DMA,
            pltpu.SemaphoreType.DMA,
            pltpu.SemaphoreType.DMA,
        ],
    )
    i3 = edge_index[0].reshape(NW, n_ch, CH)
    j3 = edge_index[1].reshape(NW, n_ch, CH)
    gx = gather_fn(t_src, t_dst, i3, j3)

    ewt = edge_w.T
    be = 16000
    ne = E // be
    outt = pl.pallas_call(
        functools.partial(_mlp_body, float(cat_dim)),
        grid=(ne,),
        in_specs=[
            pl.BlockSpec((ED, be), lambda b: (0, b)),
            pl.BlockSpec((24, be), lambda b: (0, b)),
            pl.BlockSpec((ED, 2 * ED), lambda b: (0, 0)),
            pl.BlockSpec((ED, 8), lambda b: (0, 0)),
        ],
        out_specs=pl.BlockSpec((ED, be), lambda b: (0, b)),
        out_shape=jax.ShapeDtypeStruct((ED, E), jnp.float32),
    )(ewt, gx, cw, uvb)
    return outt.T
